# Initial kernel scaffold; baseline (speedup 1.0000x reference)
import jax
import jax.numpy as jnp
from jax.experimental import pallas as pl


def kernel(query_pos, query_x, input_pos, input_x, time_emb, W_r1, b_r1, W_r2, b_r2, W_val, W_alpha, W_sh, W_skip, b_out, query_b):
    raise NotImplementedError("write your pallas kernel here")


# speedup vs baseline: 3.0223x; 3.0223x over previous
import functools
import math

import jax
import jax.numpy as jnp
import numpy as np
from jax import lax
from jax.experimental import pallas as pl
from jax.experimental.pallas import tpu as pltpu
from jax.experimental.pallas import tpu_sc as plsc

N_Q = 10000
N_IN = 10000
K = 16
D = 128
H = 8
HD = 16
LEN_DIM = 32
TIME_DIM = 32
FC = 64
R_MAX = 0.5
R_MIN = 0.05

QB = 200
NBLK = N_Q // QB
QP = 10240
TW = 128

_CHUNK = 128
_NCHUNK = (K * QP) // _CHUNK
_NW = 32
_PER_W = _NCHUNK // _NW

def _freqs():
    half = LEN_DIM // 2
    i = lax.broadcasted_iota(jnp.int32, (1, half), 1).astype(jnp.float32)
    return jnp.exp(-math.log(10000.0) * i / (half - 1))


def _prep_body(ix_ref, qx_ref, wval_ref, wskip_ref, bout_ref,
               temb_ref, wr1_ref, br1_ref, table_ref, skip_ref, pre1_ref):
    table_ref[...] = jnp.dot(ix_ref[...], wval_ref[...],
                             preferred_element_type=jnp.float32)
    skip_ref[...] = (
        jnp.dot(qx_ref[...], wskip_ref[...], preferred_element_type=jnp.float32)
        + bout_ref[...])
    pre1_ref[...] = (
        jnp.dot(temb_ref[...], wr1_ref[LEN_DIM:, :],
                preferred_element_type=jnp.float32)
        + br1_ref[...])


def _knn_body(qpos_ref, ipt_ref, idx_ref, d2_ref, px_ref, py_ref, pz_ref):
    qp = qpos_ref[...]
    ipt = ipt_ref[...]
    s = None
    for c in range(3):
        dc = ipt[c:c + 1, :] - qp[:, c:c + 1]
        s = dc * dc if s is None else s + dc * dc
    iota = lax.broadcasted_iota(jnp.int32, (QB, N_IN), 1)
    for k in range(K):
        m = jnp.min(s, axis=1, keepdims=True)
        cand = jnp.where(s == m, iota, N_IN)
        ib = jnp.min(cand, axis=1, keepdims=True)
        sel = iota == ib
        idx_ref[:, k:k + 1] = ib
        d2_ref[:, k:k + 1] = m
        px_ref[:, k:k + 1] = jnp.sum(
            jnp.where(sel, ipt[0:1, :], 0.0), axis=1, keepdims=True)
        py_ref[:, k:k + 1] = jnp.sum(
            jnp.where(sel, ipt[1:2, :], 0.0), axis=1, keepdims=True)
        pz_ref[:, k:k + 1] = jnp.sum(
            jnp.where(sel, ipt[2:3, :], 0.0), axis=1, keepdims=True)
        s = jnp.where(sel, jnp.float32(jnp.inf), s)


@functools.lru_cache(maxsize=1)
def _sc_gather_fn():
    mesh = plsc.VectorSubcoreMesh(core_axis_name="c", subcore_axis_name="s")

    @functools.partial(
        pl.kernel, mesh=mesh,
        out_type=jax.ShapeDtypeStruct((K * QP, TW), jnp.float32),
        scratch_types=[
            pltpu.VMEM((_CHUNK,), jnp.int32),
            pltpu.VMEM((_CHUNK, TW), jnp.float32),
            pltpu.SemaphoreType.DMA,
        ],
    )
    def gather(table_hbm, idx_hbm, out_hbm, idx_v, rows_v, sem):
        wid = lax.axis_index("s") * 2 + lax.axis_index("c")

        def body(j, carry):
            base = (wid * _PER_W + j) * _CHUNK
            pltpu.sync_copy(idx_hbm.at[pl.ds(base, _CHUNK)], idx_v)
            pltpu.async_copy(table_hbm.at[idx_v], rows_v, sem).wait()
            pltpu.sync_copy(rows_v, out_hbm.at[pl.ds(base, _CHUNK)])
            return carry

        lax.fori_loop(0, _PER_W, body, 0)

    return gather


def _edge_body(g_ref, d2_ref, px_ref, py_ref, pz_ref, qpos_ref, skip_ref,
               w1l_ref, pre1_ref, wr2_ref, br2_ref, a_ref, wsht_ref,
               bexp_ref, out_ref):
    qp = qpos_ref[...]
    w1l = w1l_ref[...]
    pre1 = pre1_ref[...]
    wr2 = wr2_ref[...]
    br2 = br2_ref[...]
    amat = a_ref[...]
    wsht = wsht_ref[...]
    bexp = bexp_ref[...]
    freqs = _freqs()

    m = jnp.full((QB, H), -1e30, jnp.float32)
    ssum = jnp.zeros((QB, H), jnp.float32)
    acc = jnp.zeros((QB, D), jnp.float32)
    for k in range(K):
        gk = g_ref[k]
        r2 = d2_ref[:, k:k + 1]
        r = jnp.sqrt(r2 + 1e-12)
        pos = jnp.concatenate(
            [px_ref[:, k:k + 1], py_ref[:, k:k + 1], pz_ref[:, k:k + 1]],
            axis=1)
        vec = pos - qp
        unit = vec / r
        sh = jnp.concatenate([jnp.ones((QB, 1), jnp.float32), unit], axis=1)
        inrange = r < R_MAX
        w_edge = jnp.where(inrange, 0.5 * (jnp.cos(jnp.pi * r / R_MAX) + 1.0), 0.0)
        w_edge = w_edge * jax.nn.sigmoid((r - R_MIN) / (0.1 * R_MIN))
        ang = r * freqs
        le = jnp.concatenate([jnp.sin(ang), jnp.cos(ang)], axis=1)
        h1 = jax.nn.relu(
            jnp.dot(le, w1l, preferred_element_type=jnp.float32) + pre1)
        radial = jnp.dot(h1, wr2, preferred_element_type=jnp.float32) + br2
        v = gk * jax.nn.silu(radial)
        logit = jax.nn.leaky_relu(
            jnp.dot(v, amat, preferred_element_type=jnp.float32), 0.2)
        logit = logit + jnp.dot(sh, wsht, preferred_element_type=jnp.float32)
        logit = jnp.where(inrange, logit, -1e9)
        mn = jnp.maximum(m, logit)
        cold = jnp.exp(m - mn)
        p = jnp.exp(logit - mn)
        ssum = ssum * cold + p
        wk = p * w_edge
        cexp = jnp.dot(cold, bexp, preferred_element_type=jnp.float32)
        wexp = jnp.dot(wk, bexp, preferred_element_type=jnp.float32)
        acc = acc * cexp + v * wexp
        m = mn
    sexp = jnp.dot(ssum, bexp, preferred_element_type=jnp.float32)
    out_ref[...] = acc / (sexp + 1e-9) + skip_ref[...]


def _sc_gather(table, idx_flat):
    return _sc_gather_fn()(table, idx_flat)


def kernel(query_pos, query_x, input_pos, input_x, time_emb, W_r1, b_r1,
           W_r2, b_r2, W_val, W_alpha, W_sh, W_skip, b_out, query_b):
    f32 = jnp.float32
    table, skip, pre1 = pl.pallas_call(
        _prep_body,
        out_shape=[
            jax.ShapeDtypeStruct((N_IN, TW), f32),
            jax.ShapeDtypeStruct((N_Q, D), f32),
            jax.ShapeDtypeStruct((1, FC), f32),
        ],
    )(input_x, query_x, W_val, W_skip, b_out.reshape(1, D),
      time_emb, W_r1, b_r1.reshape(1, FC))

    idx, d2, px, py, pz = pl.pallas_call(
        _knn_body,
        grid=(NBLK,),
        in_specs=[
            pl.BlockSpec((QB, 3), lambda i: (i, 0)),
            pl.BlockSpec((3, N_IN), lambda i: (0, 0)),
        ],
        out_specs=[pl.BlockSpec((QB, K), lambda i: (i, 0))] * 5,
        out_shape=[
            jax.ShapeDtypeStruct((N_Q, K), jnp.int32),
            jax.ShapeDtypeStruct((N_Q, K), f32),
            jax.ShapeDtypeStruct((N_Q, K), f32),
            jax.ShapeDtypeStruct((N_Q, K), f32),
            jax.ShapeDtypeStruct((N_Q, K), f32),
        ],
    )(query_pos, input_pos.T)

    idx_pad = jnp.zeros((K, QP), jnp.int32).at[:, :N_Q].set(idx.T).reshape(-1)
    g3 = _sc_gather(table, idx_pad).reshape(K, QP, TW)

    amat = (W_alpha[:, :, None] * jnp.eye(H, dtype=f32)[:, None, :]).reshape(D, H)
    bexp = jnp.kron(jnp.eye(H, dtype=f32), jnp.ones((1, HD), f32))

    out = pl.pallas_call(
        _edge_body,
        grid=(NBLK,),
        in_specs=[
            pl.BlockSpec((K, QB, TW), lambda i: (0, i, 0)),
            pl.BlockSpec((QB, K), lambda i: (i, 0)),
            pl.BlockSpec((QB, K), lambda i: (i, 0)),
            pl.BlockSpec((QB, K), lambda i: (i, 0)),
            pl.BlockSpec((QB, K), lambda i: (i, 0)),
            pl.BlockSpec((QB, 3), lambda i: (i, 0)),
            pl.BlockSpec((QB, D), lambda i: (i, 0)),
            pl.BlockSpec((LEN_DIM, FC), lambda i: (0, 0)),
            pl.BlockSpec((1, FC), lambda i: (0, 0)),
            pl.BlockSpec((FC, D), lambda i: (0, 0)),
            pl.BlockSpec((1, D), lambda i: (0, 0)),
            pl.BlockSpec((D, H), lambda i: (0, 0)),
            pl.BlockSpec((4, H), lambda i: (0, 0)),
            pl.BlockSpec((H, D), lambda i: (0, 0)),
        ],
        out_specs=pl.BlockSpec((QB, D), lambda i: (i, 0)),
        out_shape=jax.ShapeDtypeStruct((N_Q, D), f32),
    )(g3, d2, px, py, pz, query_pos, skip, W_r1[:LEN_DIM], pre1, W_r2,
      b_r2.reshape(1, D), amat, W_sh.T, bexp)
    return out


# speedup vs baseline: 5.9962x; 1.9840x over previous
import functools
import math

import jax
import jax.numpy as jnp
import numpy as np
from jax import lax
from jax.experimental import pallas as pl
from jax.experimental.pallas import tpu as pltpu
from jax.experimental.pallas import tpu_sc as plsc

N_Q = 10000
N_IN = 10000
K = 16
D = 128
H = 8
HD = 16
LEN_DIM = 32
TIME_DIM = 32
FC = 64
R_MAX = 0.5
R_MIN = 0.05

QB = 200
NBLK = N_Q // QB
QBE = 400
NBLKE = N_Q // QBE
EK = K * QBE
QP = 10240
TW = 256

_CHUNK = 128
_NCHUNK = (K * QP) // _CHUNK
_NW = 32
_PER_W = _NCHUNK // _NW

def _freqs():
    half = LEN_DIM // 2
    i = lax.broadcasted_iota(jnp.int32, (1, half), 1).astype(jnp.float32)
    return jnp.exp(-math.log(10000.0) * i / (half - 1))


def _prep_body(ipos_ref, ix_ref, qx_ref, wval_ref, wskip_ref, bout_ref,
               temb_ref, wr1_ref, br1_ref, table_ref, skip_ref, pre1_ref):
    table_ref[:, 0:D] = jnp.dot(ix_ref[...], wval_ref[...],
                                preferred_element_type=jnp.float32)
    table_ref[:, D:D + 8] = jnp.concatenate(
        [ipos_ref[...], jnp.zeros((N_IN, 5), jnp.float32)], axis=1)
    skip_ref[...] = (
        jnp.dot(qx_ref[...], wskip_ref[...], preferred_element_type=jnp.float32)
        + bout_ref[...])
    pre1_ref[...] = (
        jnp.dot(temb_ref[...], wr1_ref[LEN_DIM:, :],
                preferred_element_type=jnp.float32)
        + br1_ref[...])


def _knn_body(qpos_ref, ipt_ref, idx_ref, d2_ref):
    qp = qpos_ref[...]
    ipt = ipt_ref[...]
    s = None
    for c in range(3):
        dc = ipt[c:c + 1, :] - qp[:, c:c + 1]
        s = dc * dc if s is None else s + dc * dc
    iota = lax.broadcasted_iota(jnp.int32, (QB, N_IN), 1)
    for k in range(K):
        m = jnp.min(s, axis=1, keepdims=True)
        cand = jnp.where(s == m, iota, N_IN)
        ib = jnp.min(cand, axis=1, keepdims=True)
        idx_ref[:, k:k + 1] = ib
        d2_ref[:, k:k + 1] = m
        s = jnp.where(iota == ib, jnp.float32(jnp.inf), s)


@functools.lru_cache(maxsize=1)
def _sc_gather_fn():
    mesh = plsc.VectorSubcoreMesh(core_axis_name="c", subcore_axis_name="s")

    @functools.partial(
        pl.kernel, mesh=mesh,
        out_type=jax.ShapeDtypeStruct((K * QP, TW), jnp.float32),
        scratch_types=[
            pltpu.VMEM((_CHUNK,), jnp.int32),
            pltpu.VMEM((_CHUNK, TW), jnp.float32),
            pltpu.SemaphoreType.DMA,
        ],
    )
    def gather(table_hbm, idx_hbm, out_hbm, idx_v, rows_v, sem):
        wid = lax.axis_index("s") * 2 + lax.axis_index("c")

        def body(j, carry):
            base = (wid * _PER_W + j) * _CHUNK
            pltpu.sync_copy(idx_hbm.at[pl.ds(base, _CHUNK)], idx_v)
            pltpu.async_copy(table_hbm.at[idx_v], rows_v, sem).wait()
            pltpu.sync_copy(rows_v, out_hbm.at[pl.ds(base, _CHUNK)])
            return carry

        lax.fori_loop(0, _PER_W, body, 0)

    return gather


def _edge_body(g_ref, d2_ref, qpos_ref, skip_ref, w1l_ref, pre1_ref, wr2_ref,
               br2_ref, a_ref, wsht_ref, bexp_ref, out_ref, le_ref, sh_ref):
    qp = qpos_ref[...]
    w1l = w1l_ref[...]
    pre1 = pre1_ref[...]
    wr2 = wr2_ref[...]
    br2 = br2_ref[...]
    amat = a_ref[...]
    wsht = wsht_ref[...]
    bexp = bexp_ref[...]
    freqs = _freqs()

    def rad(k):
        return jnp.sqrt(d2_ref[:, k:k + 1] + 1e-12)

    for k in range(K):
        r = rad(k)
        pos = g_ref[k][:, D:D + 3]
        unit = (pos - qp) / r
        sh_ref[k * QBE:(k + 1) * QBE, :] = jnp.concatenate(
            [jnp.ones((QBE, 1), jnp.float32), unit], axis=1)
        ang = r * freqs
        le_ref[k * QBE:(k + 1) * QBE, :] = jnp.concatenate(
            [jnp.sin(ang), jnp.cos(ang)], axis=1)

    h1 = jax.nn.relu(
        jnp.dot(le_ref[...], w1l, preferred_element_type=jnp.float32) + pre1)
    radial = jnp.dot(h1, wr2, preferred_element_type=jnp.float32) + br2
    g2 = g_ref[...].reshape(EK, TW)
    v = g2[:, 0:D] * jax.nn.silu(radial)
    lgv = jax.nn.leaky_relu(
        jnp.dot(v, amat, preferred_element_type=jnp.float32), 0.2)
    shl = jnp.dot(sh_ref[...], wsht, preferred_element_type=jnp.float32)
    lg = lgv + shl

    m = jnp.full((QBE, H), -1e30, jnp.float32)
    for k in range(K):
        lk = jnp.where(rad(k) < R_MAX, lg[k * QBE:(k + 1) * QBE, :], -1e9)
        m = jnp.maximum(m, lk)
    ssum = jnp.zeros((QBE, H), jnp.float32)
    acc = jnp.zeros((QBE, D), jnp.float32)
    for k in range(K):
        r = rad(k)
        inrange = r < R_MAX
        lk = jnp.where(inrange, lg[k * QBE:(k + 1) * QBE, :], -1e9)
        p = jnp.exp(lk - m)
        ssum = ssum + p
        w_edge = jnp.where(inrange, 0.5 * (jnp.cos(jnp.pi * r / R_MAX) + 1.0), 0.0)
        w_edge = w_edge * jax.nn.sigmoid((r - R_MIN) / (0.1 * R_MIN))
        wexp = jnp.dot(p * w_edge, bexp, preferred_element_type=jnp.float32)
        acc = acc + v[k * QBE:(k + 1) * QBE, :] * wexp
    sexp = jnp.dot(ssum, bexp, preferred_element_type=jnp.float32)
    out_ref[...] = acc / (sexp + 1e-9) + skip_ref[...]


def _sc_gather(table, idx_flat):
    return _sc_gather_fn()(table, idx_flat)


def kernel(query_pos, query_x, input_pos, input_x, time_emb, W_r1, b_r1,
           W_r2, b_r2, W_val, W_alpha, W_sh, W_skip, b_out, query_b):
    f32 = jnp.float32
    table, skip, pre1 = pl.pallas_call(
        _prep_body,
        out_shape=[
            jax.ShapeDtypeStruct((N_IN, TW), f32),
            jax.ShapeDtypeStruct((N_Q, D), f32),
            jax.ShapeDtypeStruct((1, FC), f32),
        ],
    )(input_pos, input_x, query_x, W_val, W_skip, b_out.reshape(1, D),
      time_emb, W_r1, b_r1.reshape(1, FC))

    idx, d2 = pl.pallas_call(
        _knn_body,
        grid=(NBLK,),
        in_specs=[
            pl.BlockSpec((QB, 3), lambda i: (i, 0)),
            pl.BlockSpec((3, N_IN), lambda i: (0, 0)),
        ],
        out_specs=[pl.BlockSpec((QB, K), lambda i: (i, 0))] * 2,
        out_shape=[
            jax.ShapeDtypeStruct((N_Q, K), jnp.int32),
            jax.ShapeDtypeStruct((N_Q, K), f32),
        ],
    )(query_pos, input_pos.T)

    idx_pad = jnp.zeros((K, QP), jnp.int32).at[:, :N_Q].set(idx.T).reshape(-1)
    g3 = _sc_gather(table, idx_pad).reshape(K, QP, TW)

    amat = (W_alpha[:, :, None] * jnp.eye(H, dtype=f32)[:, None, :]).reshape(D, H)
    bexp = jnp.kron(jnp.eye(H, dtype=f32), jnp.ones((1, HD), f32))

    out = pl.pallas_call(
        _edge_body,
        grid=(NBLKE,),
        in_specs=[
            pl.BlockSpec((K, QBE, TW), lambda i: (0, i, 0)),
            pl.BlockSpec((QBE, K), lambda i: (i, 0)),
            pl.BlockSpec((QBE, 3), lambda i: (i, 0)),
            pl.BlockSpec((QBE, D), lambda i: (i, 0)),
            pl.BlockSpec((LEN_DIM, FC), lambda i: (0, 0)),
            pl.BlockSpec((1, FC), lambda i: (0, 0)),
            pl.BlockSpec((FC, D), lambda i: (0, 0)),
            pl.BlockSpec((1, D), lambda i: (0, 0)),
            pl.BlockSpec((D, H), lambda i: (0, 0)),
            pl.BlockSpec((4, H), lambda i: (0, 0)),
            pl.BlockSpec((H, D), lambda i: (0, 0)),
        ],
        out_specs=pl.BlockSpec((QBE, D), lambda i: (i, 0)),
        out_shape=jax.ShapeDtypeStruct((N_Q, D), f32),
        scratch_shapes=[
            pltpu.VMEM((EK, LEN_DIM), f32),
            pltpu.VMEM((EK, 4), f32),
        ],
    )(g3, d2, query_pos, skip, W_r1[:LEN_DIM], pre1, W_r2,
      b_r2.reshape(1, D), amat, W_sh.T, bexp)
    return out


# speedup vs baseline: 6.2202x; 1.0374x over previous
import functools
import math

import jax
import jax.numpy as jnp
import numpy as np
from jax import lax
from jax.experimental import pallas as pl
from jax.experimental.pallas import tpu as pltpu
from jax.experimental.pallas import tpu_sc as plsc

N_Q = 10000
N_IN = 10000
K = 16
D = 128
H = 8
HD = 16
LEN_DIM = 32
TIME_DIM = 32
FC = 64
R_MAX = 0.5
R_MIN = 0.05

QH = N_Q // 2
QB = 200
NBLK = QH // QB
QBE = 200
NBLKE = QH // QBE
EK = K * QBE
QP = 5120
TW = 256

_CHUNK = 128
_NCHUNK = (K * QP) // _CHUNK
_NW = 32
_PER_W = _NCHUNK // _NW

def _freqs():
    half = LEN_DIM // 2
    i = lax.broadcasted_iota(jnp.int32, (1, half), 1).astype(jnp.float32)
    return jnp.exp(-math.log(10000.0) * i / (half - 1))


def _prep_body(ipos_ref, ix_ref, qx_ref, wval_ref, wskip_ref, bout_ref,
               temb_ref, wr1_ref, br1_ref, table_ref, skip_ref, pre1_ref):
    table_ref[:, 0:D] = jnp.dot(ix_ref[...], wval_ref[...],
                                preferred_element_type=jnp.float32)
    table_ref[:, D:D + 8] = jnp.concatenate(
        [ipos_ref[...], jnp.zeros((N_IN, 5), jnp.float32)], axis=1)
    skip_ref[...] = (
        jnp.dot(qx_ref[...], wskip_ref[...], preferred_element_type=jnp.float32)
        + bout_ref[...])
    pre1_ref[...] = (
        jnp.dot(temb_ref[...], wr1_ref[LEN_DIM:, :],
                preferred_element_type=jnp.float32)
        + br1_ref[...])


def _knn_body(qpos_ref, ipt_ref, idx_ref, d2_ref):
    qp = qpos_ref[...]
    ipt = ipt_ref[...]
    s = None
    for c in range(3):
        dc = ipt[c:c + 1, :] - qp[:, c:c + 1]
        s = dc * dc if s is None else s + dc * dc
    iota = lax.broadcasted_iota(jnp.int32, (QB, N_IN), 1)
    for k in range(K):
        m = jnp.min(s, axis=1, keepdims=True)
        cand = jnp.where(s == m, iota, N_IN)
        ib = jnp.min(cand, axis=1, keepdims=True)
        idx_ref[:, k:k + 1] = ib
        d2_ref[:, k:k + 1] = m
        s = jnp.where(iota == ib, jnp.float32(jnp.inf), s)


@functools.lru_cache(maxsize=1)
def _sc_gather_fn():
    mesh = plsc.VectorSubcoreMesh(core_axis_name="c", subcore_axis_name="s")

    @functools.partial(
        pl.kernel, mesh=mesh,
        out_type=jax.ShapeDtypeStruct((K * QP, TW), jnp.float32),
        scratch_types=[
            pltpu.VMEM((_CHUNK,), jnp.int32),
            pltpu.VMEM((_CHUNK, TW), jnp.float32),
            pltpu.SemaphoreType.DMA,
        ],
    )
    def gather(table_hbm, idx_hbm, out_hbm, idx_v, rows_v, sem):
        wid = lax.axis_index("s") * 2 + lax.axis_index("c")

        def body(j, carry):
            base = (wid * _PER_W + j) * _CHUNK
            pltpu.sync_copy(idx_hbm.at[pl.ds(base, _CHUNK)], idx_v)
            pltpu.async_copy(table_hbm.at[idx_v], rows_v, sem).wait()
            pltpu.sync_copy(rows_v, out_hbm.at[pl.ds(base, _CHUNK)])
            return carry

        lax.fori_loop(0, _PER_W, body, 0)

    return gather


def _edge_body(g_ref, d2_ref, qpos_ref, skip_ref, w1l_ref, pre1_ref, wr2_ref,
               br2_ref, a_ref, wsht_ref, bexp_ref, out_ref, le_ref, sh_ref):
    qp = qpos_ref[...]
    w1l = w1l_ref[...]
    pre1 = pre1_ref[...]
    wr2 = wr2_ref[...]
    br2 = br2_ref[...]
    amat = a_ref[...]
    wsht = wsht_ref[...]
    bexp = bexp_ref[...]
    freqs = _freqs()

    def rad(k):
        return jnp.sqrt(d2_ref[:, k:k + 1] + 1e-12)

    for k in range(K):
        r = rad(k)
        pos = g_ref[k][:, D:D + 3]
        unit = (pos - qp) / r
        sh_ref[k * QBE:(k + 1) * QBE, :] = jnp.concatenate(
            [jnp.ones((QBE, 1), jnp.float32), unit], axis=1)
        ang = r * freqs
        le_ref[k * QBE:(k + 1) * QBE, :] = jnp.concatenate(
            [jnp.sin(ang), jnp.cos(ang)], axis=1)

    h1 = jax.nn.relu(
        jnp.dot(le_ref[...], w1l, preferred_element_type=jnp.float32) + pre1)
    radial = jnp.dot(h1, wr2, preferred_element_type=jnp.float32) + br2
    g2 = g_ref[...].reshape(EK, TW)
    v = g2[:, 0:D] * jax.nn.silu(radial)
    lgv = jax.nn.leaky_relu(
        jnp.dot(v, amat, preferred_element_type=jnp.float32), 0.2)
    shl = jnp.dot(sh_ref[...], wsht, preferred_element_type=jnp.float32)
    lg = lgv + shl

    m = jnp.full((QBE, H), -1e30, jnp.float32)
    for k in range(K):
        lk = jnp.where(rad(k) < R_MAX, lg[k * QBE:(k + 1) * QBE, :], -1e9)
        m = jnp.maximum(m, lk)
    ssum = jnp.zeros((QBE, H), jnp.float32)
    acc = jnp.zeros((QBE, D), jnp.float32)
    for k in range(K):
        r = rad(k)
        inrange = r < R_MAX
        lk = jnp.where(inrange, lg[k * QBE:(k + 1) * QBE, :], -1e9)
        p = jnp.exp(lk - m)
        ssum = ssum + p
        w_edge = jnp.where(inrange, 0.5 * (jnp.cos(jnp.pi * r / R_MAX) + 1.0), 0.0)
        w_edge = w_edge * jax.nn.sigmoid((r - R_MIN) / (0.1 * R_MIN))
        wexp = jnp.dot(p * w_edge, bexp, preferred_element_type=jnp.float32)
        acc = acc + v[k * QBE:(k + 1) * QBE, :] * wexp
    sexp = jnp.dot(ssum, bexp, preferred_element_type=jnp.float32)
    out_ref[...] = acc / (sexp + 1e-9) + skip_ref[...]


def _sc_gather(table, idx_flat):
    return _sc_gather_fn()(table, idx_flat)


def kernel(query_pos, query_x, input_pos, input_x, time_emb, W_r1, b_r1,
           W_r2, b_r2, W_val, W_alpha, W_sh, W_skip, b_out, query_b):
    f32 = jnp.float32
    table, skip, pre1 = pl.pallas_call(
        _prep_body,
        out_shape=[
            jax.ShapeDtypeStruct((N_IN, TW), f32),
            jax.ShapeDtypeStruct((N_Q, D), f32),
            jax.ShapeDtypeStruct((1, FC), f32),
        ],
    )(input_pos, input_x, query_x, W_val, W_skip, b_out.reshape(1, D),
      time_emb, W_r1, b_r1.reshape(1, FC))

    amat = (W_alpha[:, :, None] * jnp.eye(H, dtype=f32)[:, None, :]).reshape(D, H)
    bexp = jnp.kron(jnp.eye(H, dtype=f32), jnp.ones((1, HD), f32))
    ipt = input_pos.T
    w1l = W_r1[:LEN_DIM]
    br2r = b_r2.reshape(1, D)
    wsht = W_sh.T

    outs = []
    for hh in range(2):
        qpos_h = query_pos[hh * QH:(hh + 1) * QH]
        skip_h = skip[hh * QH:(hh + 1) * QH]
        idx, d2 = pl.pallas_call(
            _knn_body,
            grid=(NBLK,),
            in_specs=[
                pl.BlockSpec((QB, 3), lambda i: (i, 0)),
                pl.BlockSpec((3, N_IN), lambda i: (0, 0)),
            ],
            out_specs=[pl.BlockSpec((QB, K), lambda i: (i, 0))] * 2,
            out_shape=[
                jax.ShapeDtypeStruct((QH, K), jnp.int32),
                jax.ShapeDtypeStruct((QH, K), f32),
            ],
        )(qpos_h, ipt)

        idx_pad = jnp.zeros((K, QP), jnp.int32).at[:, :QH].set(idx.T).reshape(-1)
        g3 = _sc_gather(table, idx_pad).reshape(K, QP, TW)

        out_h = pl.pallas_call(
            _edge_body,
            grid=(NBLKE,),
            in_specs=[
                pl.BlockSpec((K, QBE, TW), lambda i: (0, i, 0)),
                pl.BlockSpec((QBE, K), lambda i: (i, 0)),
                pl.BlockSpec((QBE, 3), lambda i: (i, 0)),
                pl.BlockSpec((QBE, D), lambda i: (i, 0)),
                pl.BlockSpec((LEN_DIM, FC), lambda i: (0, 0)),
                pl.BlockSpec((1, FC), lambda i: (0, 0)),
                pl.BlockSpec((FC, D), lambda i: (0, 0)),
                pl.BlockSpec((1, D), lambda i: (0, 0)),
                pl.BlockSpec((D, H), lambda i: (0, 0)),
                pl.BlockSpec((4, H), lambda i: (0, 0)),
                pl.BlockSpec((H, D), lambda i: (0, 0)),
            ],
            out_specs=pl.BlockSpec((QBE, D), lambda i: (i, 0)),
            out_shape=jax.ShapeDtypeStruct((QH, D), f32),
            scratch_shapes=[
                pltpu.VMEM((EK, LEN_DIM), f32),
                pltpu.VMEM((EK, 4), f32),
            ],
        )(g3, d2, qpos_h, skip_h, w1l, pre1, W_r2, br2r, amat, wsht, bexp)
        outs.append(out_h)
    return jnp.concatenate(outs, axis=0)


# speedup vs baseline: 6.2313x; 1.0018x over previous
import functools
import math

import jax
import jax.numpy as jnp
import numpy as np
from jax import lax
from jax.experimental import pallas as pl
from jax.experimental.pallas import tpu as pltpu
from jax.experimental.pallas import tpu_sc as plsc

N_Q = 10000
N_IN = 10000
K = 16
D = 128
H = 8
HD = 16
LEN_DIM = 32
TIME_DIM = 32
FC = 64
R_MAX = 0.5
R_MIN = 0.05

QH = N_Q // 2
QB = 200
NBLK = QH // QB
QBE = 200
NBLKE = QH // QBE
EK = K * QBE
QP = 5120
TW = 256

_CHUNK = 128
_NCHUNK = (K * QP) // _CHUNK
_NW = 32
_PER_W = _NCHUNK // _NW

def _freqs():
    half = LEN_DIM // 2
    i = lax.broadcasted_iota(jnp.int32, (1, half), 1).astype(jnp.float32)
    return jnp.exp(-math.log(10000.0) * i / (half - 1))


def _prep_body(ipos_ref, ix_ref, qx_ref, wval_ref, wskip_ref, bout_ref,
               temb_ref, wr1_ref, br1_ref, table_ref, skip_ref, pre1_ref):
    table_ref[:, 0:D] = jnp.dot(ix_ref[...], wval_ref[...],
                                preferred_element_type=jnp.float32)
    table_ref[:, D:D + 8] = jnp.concatenate(
        [ipos_ref[...], jnp.zeros((N_IN, 5), jnp.float32)], axis=1)
    skip_ref[...] = (
        jnp.dot(qx_ref[...], wskip_ref[...], preferred_element_type=jnp.float32)
        + bout_ref[...])
    pre1_ref[...] = (
        jnp.dot(temb_ref[...], wr1_ref[LEN_DIM:, :],
                preferred_element_type=jnp.float32)
        + br1_ref[...])


def _knn_body(qpos_ref, ipt_ref, idx_ref, d2_ref):
    qp = qpos_ref[...]
    ipt = ipt_ref[...]
    s = None
    for c in range(3):
        dc = ipt[c:c + 1, :] - qp[:, c:c + 1]
        s = dc * dc if s is None else s + dc * dc
    iota = lax.broadcasted_iota(jnp.int32, (QB, N_IN), 1)
    for k in range(K):
        m = jnp.min(s, axis=1, keepdims=True)
        cand = jnp.where(s == m, iota, N_IN)
        ib = jnp.min(cand, axis=1, keepdims=True)
        idx_ref[:, k:k + 1] = ib
        d2_ref[:, k:k + 1] = m
        s = jnp.where(iota == ib, jnp.float32(jnp.inf), s)


_NBUF = 2
_NSTEP = _PER_W // _NBUF


@functools.lru_cache(maxsize=1)
def _sc_gather_fn():
    mesh = plsc.VectorSubcoreMesh(core_axis_name="c", subcore_axis_name="s")

    @functools.partial(
        pl.kernel, mesh=mesh,
        out_type=jax.ShapeDtypeStruct((K * QP, TW), jnp.float32),
        scratch_types=[
            pltpu.VMEM((_PER_W * _CHUNK,), jnp.int32),
            pltpu.VMEM((_NBUF * _CHUNK, TW), jnp.float32),
            pltpu.SemaphoreType.DMA,
        ],
    )
    def gather(table_hbm, idx_hbm, out_hbm, idx_v, rows_v, sem):
        wid = lax.axis_index("s") * 2 + lax.axis_index("c")
        base_w = wid * _PER_W * _CHUNK
        pltpu.sync_copy(idx_hbm.at[pl.ds(base_w, _PER_W * _CHUNK)], idx_v)

        def step(j, carry):
            copies = []
            for b in range(_NBUF):
                copies.append(pltpu.async_copy(
                    table_hbm.at[idx_v.at[pl.ds((j * _NBUF + b) * _CHUNK,
                                                _CHUNK)]],
                    rows_v.at[pl.ds(b * _CHUNK, _CHUNK)], sem))
            for b in range(_NBUF):
                copies[b].wait()
            pltpu.sync_copy(
                rows_v,
                out_hbm.at[pl.ds(base_w + j * _NBUF * _CHUNK, _NBUF * _CHUNK)])
            return carry

        lax.fori_loop(0, _NSTEP, step, 0)

    return gather


def _edge_body(g_ref, d2_ref, qpos_ref, skip_ref, w1l_ref, pre1_ref, wr2_ref,
               br2_ref, a_ref, wsht_ref, bexp_ref, out_ref, le_ref, sh_ref):
    qp = qpos_ref[...]
    w1l = w1l_ref[...]
    pre1 = pre1_ref[...]
    wr2 = wr2_ref[...]
    br2 = br2_ref[...]
    amat = a_ref[...]
    wsht = wsht_ref[...]
    bexp = bexp_ref[...]
    freqs = _freqs()

    def rad(k):
        return jnp.sqrt(d2_ref[:, k:k + 1] + 1e-12)

    for k in range(K):
        r = rad(k)
        pos = g_ref[k][:, D:D + 3]
        unit = (pos - qp) / r
        sh_ref[k * QBE:(k + 1) * QBE, :] = jnp.concatenate(
            [jnp.ones((QBE, 1), jnp.float32), unit], axis=1)
        ang = r * freqs
        le_ref[k * QBE:(k + 1) * QBE, :] = jnp.concatenate(
            [jnp.sin(ang), jnp.cos(ang)], axis=1)

    h1 = jax.nn.relu(
        jnp.dot(le_ref[...], w1l, preferred_element_type=jnp.float32) + pre1)
    radial = jnp.dot(h1, wr2, preferred_element_type=jnp.float32) + br2
    g2 = g_ref[...].reshape(EK, TW)
    v = g2[:, 0:D] * jax.nn.silu(radial)
    lgv = jax.nn.leaky_relu(
        jnp.dot(v, amat, preferred_element_type=jnp.float32), 0.2)
    shl = jnp.dot(sh_ref[...], wsht, preferred_element_type=jnp.float32)
    lg = lgv + shl

    m = jnp.full((QBE, H), -1e30, jnp.float32)
    for k in range(K):
        lk = jnp.where(rad(k) < R_MAX, lg[k * QBE:(k + 1) * QBE, :], -1e9)
        m = jnp.maximum(m, lk)
    ssum = jnp.zeros((QBE, H), jnp.float32)
    acc = jnp.zeros((QBE, D), jnp.float32)
    for k in range(K):
        r = rad(k)
        inrange = r < R_MAX
        lk = jnp.where(inrange, lg[k * QBE:(k + 1) * QBE, :], -1e9)
        p = jnp.exp(lk - m)
        ssum = ssum + p
        w_edge = jnp.where(inrange, 0.5 * (jnp.cos(jnp.pi * r / R_MAX) + 1.0), 0.0)
        w_edge = w_edge * jax.nn.sigmoid((r - R_MIN) / (0.1 * R_MIN))
        wexp = jnp.dot(p * w_edge, bexp, preferred_element_type=jnp.float32)
        acc = acc + v[k * QBE:(k + 1) * QBE, :] * wexp
    sexp = jnp.dot(ssum, bexp, preferred_element_type=jnp.float32)
    out_ref[...] = acc / (sexp + 1e-9) + skip_ref[...]


def _sc_gather(table, idx_flat):
    return _sc_gather_fn()(table, idx_flat)


def kernel(query_pos, query_x, input_pos, input_x, time_emb, W_r1, b_r1,
           W_r2, b_r2, W_val, W_alpha, W_sh, W_skip, b_out, query_b):
    f32 = jnp.float32
    table, skip, pre1 = pl.pallas_call(
        _prep_body,
        out_shape=[
            jax.ShapeDtypeStruct((N_IN, TW), f32),
            jax.ShapeDtypeStruct((N_Q, D), f32),
            jax.ShapeDtypeStruct((1, FC), f32),
        ],
    )(input_pos, input_x, query_x, W_val, W_skip, b_out.reshape(1, D),
      time_emb, W_r1, b_r1.reshape(1, FC))

    amat = (W_alpha[:, :, None] * jnp.eye(H, dtype=f32)[:, None, :]).reshape(D, H)
    bexp = jnp.kron(jnp.eye(H, dtype=f32), jnp.ones((1, HD), f32))
    ipt = input_pos.T
    w1l = W_r1[:LEN_DIM]
    br2r = b_r2.reshape(1, D)
    wsht = W_sh.T

    outs = []
    for hh in range(2):
        qpos_h = query_pos[hh * QH:(hh + 1) * QH]
        skip_h = skip[hh * QH:(hh + 1) * QH]
        idx, d2 = pl.pallas_call(
            _knn_body,
            grid=(NBLK,),
            in_specs=[
                pl.BlockSpec((QB, 3), lambda i: (i, 0)),
                pl.BlockSpec((3, N_IN), lambda i: (0, 0)),
            ],
            out_specs=[pl.BlockSpec((QB, K), lambda i: (i, 0))] * 2,
            out_shape=[
                jax.ShapeDtypeStruct((QH, K), jnp.int32),
                jax.ShapeDtypeStruct((QH, K), f32),
            ],
        )(qpos_h, ipt)

        idx_pad = jnp.zeros((K, QP), jnp.int32).at[:, :QH].set(idx.T).reshape(-1)
        g3 = _sc_gather(table, idx_pad).reshape(K, QP, TW)

        out_h = pl.pallas_call(
            _edge_body,
            grid=(NBLKE,),
            in_specs=[
                pl.BlockSpec((K, QBE, TW), lambda i: (0, i, 0)),
                pl.BlockSpec((QBE, K), lambda i: (i, 0)),
                pl.BlockSpec((QBE, 3), lambda i: (i, 0)),
                pl.BlockSpec((QBE, D), lambda i: (i, 0)),
                pl.BlockSpec((LEN_DIM, FC), lambda i: (0, 0)),
                pl.BlockSpec((1, FC), lambda i: (0, 0)),
                pl.BlockSpec((FC, D), lambda i: (0, 0)),
                pl.BlockSpec((1, D), lambda i: (0, 0)),
                pl.BlockSpec((D, H), lambda i: (0, 0)),
                pl.BlockSpec((4, H), lambda i: (0, 0)),
                pl.BlockSpec((H, D), lambda i: (0, 0)),
            ],
            out_specs=pl.BlockSpec((QBE, D), lambda i: (i, 0)),
            out_shape=jax.ShapeDtypeStruct((QH, D), f32),
            scratch_shapes=[
                pltpu.VMEM((EK, LEN_DIM), f32),
                pltpu.VMEM((EK, 4), f32),
            ],
        )(g3, d2, qpos_h, skip_h, w1l, pre1, W_r2, br2r, amat, wsht, bexp)
        outs.append(out_h)
    return jnp.concatenate(outs, axis=0)


# speedup vs baseline: 6.5011x; 1.0433x over previous
import functools
import math

import jax
import jax.numpy as jnp
import numpy as np
from jax import lax
from jax.experimental import pallas as pl
from jax.experimental.pallas import tpu as pltpu
from jax.experimental.pallas import tpu_sc as plsc

N_Q = 10000
N_IN = 10000
K = 16
D = 128
H = 8
HD = 16
LEN_DIM = 32
TIME_DIM = 32
FC = 64
R_MAX = 0.5
R_MIN = 0.05

QH = N_Q // 2
QB = 200
NBLK = QH // QB
QBE = 200
NBLKE = QH // QBE
EK = K * QBE
QP = 5120
TW = 256

_CHUNK = 128
_NCHUNK = (K * QP) // _CHUNK
_NW = 32
_PER_W = _NCHUNK // _NW

def _freqs():
    half = LEN_DIM // 2
    i = lax.broadcasted_iota(jnp.int32, (1, half), 1).astype(jnp.float32)
    return jnp.exp(-math.log(10000.0) * i / (half - 1))


def _prep_body(ipos_ref, ix_ref, qx_ref, wval_ref, wskip_ref, bout_ref,
               temb_ref, wr1_ref, br1_ref, table_ref, skip_ref, pre1_ref):
    table_ref[:, 0:D] = jnp.dot(ix_ref[...], wval_ref[...],
                                preferred_element_type=jnp.float32)
    table_ref[:, D:D + 8] = jnp.concatenate(
        [ipos_ref[...], jnp.zeros((N_IN, 5), jnp.float32)], axis=1)
    skip_ref[...] = (
        jnp.dot(qx_ref[...], wskip_ref[...], preferred_element_type=jnp.float32)
        + bout_ref[...])
    pre1_ref[...] = (
        jnp.dot(temb_ref[...], wr1_ref[LEN_DIM:, :],
                preferred_element_type=jnp.float32)
        + br1_ref[...])


def _knn_body(qpos_ref, ipt_ref, idx_ref, d2_ref):
    qp = qpos_ref[...]
    ipt = ipt_ref[...]
    s = None
    for c in range(3):
        dc = ipt[c:c + 1, :] - qp[:, c:c + 1]
        s = dc * dc if s is None else s + dc * dc
    iota = lax.broadcasted_iota(jnp.int32, (QB, N_IN), 1)
    m = jnp.min(s, axis=1, keepdims=True)
    for k in range(K):
        cand = jnp.where(s == m, iota, N_IN)
        ib = jnp.min(cand, axis=1, keepdims=True)
        idx_ref[:, k:k + 1] = ib
        d2_ref[:, k:k + 1] = m
        if k < K - 1:
            m = jnp.min(jnp.where(s > m, s, jnp.float32(jnp.inf)),
                        axis=1, keepdims=True)


_NBUF = 2
_NSTEP = _PER_W // _NBUF


@functools.lru_cache(maxsize=1)
def _sc_gather_fn():
    mesh = plsc.VectorSubcoreMesh(core_axis_name="c", subcore_axis_name="s")

    @functools.partial(
        pl.kernel, mesh=mesh,
        out_type=jax.ShapeDtypeStruct((K * QP, TW), jnp.float32),
        scratch_types=[
            pltpu.VMEM((_PER_W * _CHUNK,), jnp.int32),
            pltpu.VMEM((_NBUF * _CHUNK, TW), jnp.float32),
            pltpu.SemaphoreType.DMA,
        ],
    )
    def gather(table_hbm, idx_hbm, out_hbm, idx_v, rows_v, sem):
        wid = lax.axis_index("s") * 2 + lax.axis_index("c")
        base_w = wid * _PER_W * _CHUNK
        pltpu.sync_copy(idx_hbm.at[pl.ds(base_w, _PER_W * _CHUNK)], idx_v)

        def step(j, carry):
            copies = []
            for b in range(_NBUF):
                copies.append(pltpu.async_copy(
                    table_hbm.at[idx_v.at[pl.ds((j * _NBUF + b) * _CHUNK,
                                                _CHUNK)]],
                    rows_v.at[pl.ds(b * _CHUNK, _CHUNK)], sem))
            for b in range(_NBUF):
                copies[b].wait()
            pltpu.sync_copy(
                rows_v,
                out_hbm.at[pl.ds(base_w + j * _NBUF * _CHUNK, _NBUF * _CHUNK)])
            return carry

        lax.fori_loop(0, _NSTEP, step, 0)

    return gather


def _edge_body(g_ref, d2_ref, qpos_ref, skip_ref, w1l_ref, pre1_ref, wr2_ref,
               br2_ref, a_ref, wsht_ref, bexp_ref, out_ref, le_ref, sh_ref):
    qp = qpos_ref[...]
    w1l = w1l_ref[...]
    pre1 = pre1_ref[...]
    wr2 = wr2_ref[...]
    br2 = br2_ref[...]
    amat = a_ref[...]
    wsht = wsht_ref[...]
    bexp = bexp_ref[...]
    freqs = _freqs()

    def rad(k):
        return jnp.sqrt(d2_ref[:, k:k + 1] + 1e-12)

    for k in range(K):
        r = rad(k)
        pos = g_ref[k][:, D:D + 3]
        unit = (pos - qp) / r
        sh_ref[k * QBE:(k + 1) * QBE, :] = jnp.concatenate(
            [jnp.ones((QBE, 1), jnp.float32), unit], axis=1)
        ang = r * freqs
        le_ref[k * QBE:(k + 1) * QBE, :] = jnp.concatenate(
            [jnp.sin(ang), jnp.cos(ang)], axis=1)

    h1 = jax.nn.relu(
        jnp.dot(le_ref[...], w1l, preferred_element_type=jnp.float32) + pre1)
    radial = jnp.dot(h1, wr2, preferred_element_type=jnp.float32) + br2
    g2 = g_ref[...].reshape(EK, TW)
    v = g2[:, 0:D] * jax.nn.silu(radial)
    lgv = jax.nn.leaky_relu(
        jnp.dot(v, amat, preferred_element_type=jnp.float32), 0.2)
    shl = jnp.dot(sh_ref[...], wsht, preferred_element_type=jnp.float32)
    lg = lgv + shl

    m = jnp.full((QBE, H), -1e30, jnp.float32)
    for k in range(K):
        lk = jnp.where(rad(k) < R_MAX, lg[k * QBE:(k + 1) * QBE, :], -1e9)
        m = jnp.maximum(m, lk)
    ssum = jnp.zeros((QBE, H), jnp.float32)
    acc = jnp.zeros((QBE, D), jnp.float32)
    for k in range(K):
        r = rad(k)
        inrange = r < R_MAX
        lk = jnp.where(inrange, lg[k * QBE:(k + 1) * QBE, :], -1e9)
        p = jnp.exp(lk - m)
        ssum = ssum + p
        w_edge = jnp.where(inrange, 0.5 * (jnp.cos(jnp.pi * r / R_MAX) + 1.0), 0.0)
        w_edge = w_edge * jax.nn.sigmoid((r - R_MIN) / (0.1 * R_MIN))
        wexp = jnp.dot(p * w_edge, bexp, preferred_element_type=jnp.float32)
        acc = acc + v[k * QBE:(k + 1) * QBE, :] * wexp
    sexp = jnp.dot(ssum, bexp, preferred_element_type=jnp.float32)
    out_ref[...] = acc / (sexp + 1e-9) + skip_ref[...]


def _sc_gather(table, idx_flat):
    return _sc_gather_fn()(table, idx_flat)


def kernel(query_pos, query_x, input_pos, input_x, time_emb, W_r1, b_r1,
           W_r2, b_r2, W_val, W_alpha, W_sh, W_skip, b_out, query_b):
    f32 = jnp.float32
    table, skip, pre1 = pl.pallas_call(
        _prep_body,
        out_shape=[
            jax.ShapeDtypeStruct((N_IN, TW), f32),
            jax.ShapeDtypeStruct((N_Q, D), f32),
            jax.ShapeDtypeStruct((1, FC), f32),
        ],
    )(input_pos, input_x, query_x, W_val, W_skip, b_out.reshape(1, D),
      time_emb, W_r1, b_r1.reshape(1, FC))

    amat = (W_alpha[:, :, None] * jnp.eye(H, dtype=f32)[:, None, :]).reshape(D, H)
    bexp = jnp.kron(jnp.eye(H, dtype=f32), jnp.ones((1, HD), f32))
    ipt = input_pos.T
    w1l = W_r1[:LEN_DIM]
    br2r = b_r2.reshape(1, D)
    wsht = W_sh.T

    outs = []
    for hh in range(2):
        qpos_h = query_pos[hh * QH:(hh + 1) * QH]
        skip_h = skip[hh * QH:(hh + 1) * QH]
        idx, d2 = pl.pallas_call(
            _knn_body,
            grid=(NBLK,),
            in_specs=[
                pl.BlockSpec((QB, 3), lambda i: (i, 0)),
                pl.BlockSpec((3, N_IN), lambda i: (0, 0)),
            ],
            out_specs=[pl.BlockSpec((QB, K), lambda i: (i, 0))] * 2,
            out_shape=[
                jax.ShapeDtypeStruct((QH, K), jnp.int32),
                jax.ShapeDtypeStruct((QH, K), f32),
            ],
        )(qpos_h, ipt)

        idx_pad = jnp.zeros((K, QP), jnp.int32).at[:, :QH].set(idx.T).reshape(-1)
        g3 = _sc_gather(table, idx_pad).reshape(K, QP, TW)

        out_h = pl.pallas_call(
            _edge_body,
            grid=(NBLKE,),
            in_specs=[
                pl.BlockSpec((K, QBE, TW), lambda i: (0, i, 0)),
                pl.BlockSpec((QBE, K), lambda i: (i, 0)),
                pl.BlockSpec((QBE, 3), lambda i: (i, 0)),
                pl.BlockSpec((QBE, D), lambda i: (i, 0)),
                pl.BlockSpec((LEN_DIM, FC), lambda i: (0, 0)),
                pl.BlockSpec((1, FC), lambda i: (0, 0)),
                pl.BlockSpec((FC, D), lambda i: (0, 0)),
                pl.BlockSpec((1, D), lambda i: (0, 0)),
                pl.BlockSpec((D, H), lambda i: (0, 0)),
                pl.BlockSpec((4, H), lambda i: (0, 0)),
                pl.BlockSpec((H, D), lambda i: (0, 0)),
            ],
            out_specs=pl.BlockSpec((QBE, D), lambda i: (i, 0)),
            out_shape=jax.ShapeDtypeStruct((QH, D), f32),
            scratch_shapes=[
                pltpu.VMEM((EK, LEN_DIM), f32),
                pltpu.VMEM((EK, 4), f32),
            ],
        )(g3, d2, qpos_h, skip_h, w1l, pre1, W_r2, br2r, amat, wsht, bexp)
        outs.append(out_h)
    return jnp.concatenate(outs, axis=0)
